# baseline (speedup 1.0000x reference)
import functools
import math

import jax
import jax.numpy as jnp
from jax import lax
from jax.experimental import pallas as pl
from jax.experimental.pallas import tpu as pltpu
from jax.experimental.pallas import tpu_sc as plsc

T = 2048
D = 1024
H = 2048
E = 8
BM = 128
BM_LOG2 = 7
B_MAX = (T * 2) // BM + E
M_PAD = B_MAX * BM
BH = 512
NH = H // BH
NW = 32
TPW = T // NW
CH = 16


def _router_body(x_ref, rw_ref, probs_ref, pos0_ref, pos1_ref,
                 w0b_ref, w1b_ref, be_ref, slot_ref, nxt_ref):
    x = x_ref[...]
    logits = lax.dot_general(x, rw_ref[...], (((1,), (1,)), ((), ())),
                             preferred_element_type=jnp.float32)
    mx = jnp.max(logits, axis=1, keepdims=True)
    ex = jnp.exp(logits - mx)
    probs = ex / jnp.sum(ex, axis=1, keepdims=True)
    probs_ref[...] = probs

    e_io = lax.broadcasted_iota(jnp.int32, (T, E), 1)
    v1 = jnp.max(probs, axis=1, keepdims=True)
    i1 = jnp.min(jnp.where(probs == v1, e_io, E), axis=1, keepdims=True)
    pmask = jnp.where(e_io == i1, -1.0, probs)
    v2 = jnp.max(pmask, axis=1, keepdims=True)
    i2 = jnp.min(jnp.where(pmask == v2, e_io, E), axis=1, keepdims=True)
    den = v1 + v2
    w0b_ref[...] = jnp.broadcast_to(v1 / den, (T, 16))
    w1b_ref[...] = jnp.broadcast_to(v2 / den, (T, 16))

    h1 = (e_io == i1).astype(jnp.float32)
    h2 = (e_io == i2).astype(jnp.float32)
    eye = (lax.broadcasted_iota(jnp.int32, (E, E), 0) ==
           lax.broadcasted_iota(jnp.int32, (E, E), 1)).astype(jnp.float32)
    tdims = (((1,), (1,)), ((), ()))
    g1 = lax.dot_general(eye, h1, tdims, preferred_element_type=jnp.float32)
    g2 = lax.dot_general(eye, h2, tdims, preferred_element_type=jnp.float32)

    def cumsum_lanes(g):
        s = 1
        while s < T:
            g = g + jnp.concatenate(
                [jnp.zeros((E, s), jnp.float32), g[:, :T - s]], axis=1)
            s *= 2
        return g

    c1 = cumsum_lanes(g1)
    c2 = cumsum_lanes(g2)
    c1t = c1[:, T - 1:T]
    c2t = c2[:, T - 1:T]
    counts = (c1t + c2t).astype(jnp.int32)
    nb = lax.shift_right_logical(counts + (BM - 1), BM_LOG2)

    def cumsum_sub(v):
        s = 1
        while s < E:
            v = v + jnp.concatenate(
                [jnp.zeros((s, 1), v.dtype), v[:E - s, :]], axis=0)
            s *= 2
        return v

    off_blk = cumsum_sub(nb) - nb
    off_pad = (off_blk * BM).astype(jnp.float32)
    r0 = c1 - g1
    r1 = c1t + c2 - g2
    pos0_ref[...] = jnp.sum(g1 * (off_pad + r0), axis=0,
                            keepdims=True).astype(jnp.int32)
    pos1_ref[...] = jnp.sum(g2 * (off_pad + r1), axis=0,
                            keepdims=True).astype(jnp.int32)

    m_io = lax.broadcasted_iota(jnp.int32, (E, B_MAX), 1)
    lo = jnp.broadcast_to(off_blk, (E, B_MAX))
    hi = lo + jnp.broadcast_to(nb, (E, B_MAX))
    e_col = lax.broadcasted_iota(jnp.int32, (E, B_MAX), 0)
    ind01 = ((m_io >= lo) & (m_io < hi)).astype(jnp.int32)
    bev = jnp.sum(ind01 * (e_col + 1), axis=0, keepdims=True) - 1
    be_ref[...] = bev

    be_prev = jnp.concatenate(
        [jnp.full((1, 1), -7, jnp.int32), bev[:, :B_MAX - 1]], axis=1)
    active = bev >= 0
    run_start = (active & (bev != be_prev)).astype(jnp.float32)
    s = 1
    while s < B_MAX:
        run_start = run_start + jnp.concatenate(
            [jnp.zeros((1, s), jnp.float32), run_start[:, :B_MAX - s]], axis=1)
        s *= 2
    slot_ref[...] = jnp.bitwise_and(run_start.astype(jnp.int32) - 1, 1)

    nbm = jnp.sum(ind01 * jnp.broadcast_to(nb, (E, B_MAX)), axis=0,
                  keepdims=True)
    m_row = lax.broadcasted_iota(jnp.int32, (1, B_MAX), 1)
    nxt_idx = m_row + nbm
    tb = off_blk[E - 1:E, :] + nb[E - 1:E, :]
    ci = jnp.minimum(nxt_idx, B_MAX - 1)
    pmat = (lax.broadcasted_iota(jnp.int32, (B_MAX, B_MAX), 0) ==
            jnp.broadcast_to(ci, (B_MAX, B_MAX))).astype(jnp.float32)
    nxtv = lax.dot_general(bev.astype(jnp.float32), pmat,
                           (((1,), (0,)), ((), ())),
                           preferred_element_type=jnp.float32)
    nxt_ref[...] = jnp.where(active & (nxt_idx < tb),
                             nxtv.astype(jnp.int32), -1)


_router = pl.pallas_call(
    _router_body,
    out_shape=[
        jax.ShapeDtypeStruct((T, E), jnp.float32),
        jax.ShapeDtypeStruct((1, T), jnp.int32),
        jax.ShapeDtypeStruct((1, T), jnp.int32),
        jax.ShapeDtypeStruct((T, 16), jnp.float32),
        jax.ShapeDtypeStruct((T, 16), jnp.float32),
        jax.ShapeDtypeStruct((1, B_MAX), jnp.int32),
        jax.ShapeDtypeStruct((1, B_MAX), jnp.int32),
        jax.ShapeDtypeStruct((1, B_MAX), jnp.int32),
    ],
)


@functools.cache
def _sc_kernels():
    mesh = plsc.VectorSubcoreMesh(core_axis_name="c", subcore_axis_name="s")

    @functools.partial(
        pl.kernel,
        mesh=mesh,
        out_type=jax.ShapeDtypeStruct((M_PAD, D), jnp.float32),
        scratch_types=[
            pltpu.VMEM((TPW,), jnp.int32),
            pltpu.VMEM((TPW,), jnp.int32),
            pltpu.VMEM((TPW, D), jnp.float32),
            pltpu.SemaphoreType.DMA,
        ],
    )
    def _gather_sc(x_hbm, pos0_hbm, pos1_hbm, xs_hbm,
                   idx0_v, idx1_v, rows_v, sem):
        wid = lax.axis_index("s") * 2 + lax.axis_index("c")
        t0 = wid * TPW
        pltpu.sync_copy(pos0_hbm.at[pl.ds(t0, TPW)], idx0_v)
        pltpu.sync_copy(pos1_hbm.at[pl.ds(t0, TPW)], idx1_v)
        pltpu.sync_copy(x_hbm.at[pl.ds(t0, TPW)], rows_v)
        pltpu.async_copy(rows_v, xs_hbm.at[idx0_v], sem).wait()
        pltpu.async_copy(rows_v, xs_hbm.at[idx1_v], sem).wait()

    @functools.partial(
        pl.kernel,
        mesh=mesh,
        out_type=jax.ShapeDtypeStruct((T, D), jnp.float32),
        scratch_types=[
            pltpu.VMEM((TPW,), jnp.int32),
            pltpu.VMEM((TPW,), jnp.int32),
            pltpu.VMEM((TPW, 16), jnp.float32),
            pltpu.VMEM((TPW, 16), jnp.float32),
            pltpu.VMEM((CH, D), jnp.float32),
            pltpu.VMEM((CH, D), jnp.float32),
            pltpu.VMEM((CH, D), jnp.float32),
            pltpu.VMEM((CH, D), jnp.float32),
            pltpu.VMEM((CH, D), jnp.float32),
            pltpu.VMEM((CH, D), jnp.float32),
            pltpu.SemaphoreType.DMA,
            pltpu.SemaphoreType.DMA,
            pltpu.SemaphoreType.DMA,
            pltpu.SemaphoreType.DMA,
            pltpu.SemaphoreType.DMA,
            pltpu.SemaphoreType.DMA,
        ],
    )
    def _combine_sc(ys_hbm, pos0_hbm, pos1_hbm, w0b_hbm, w1b_hbm, out_hbm,
                    idx0_v, idx1_v, w0_v, w1_v,
                    r0a, r1a, oa, r0b, r1b, ob,
                    s0a, s1a, soa, s0b, s1b, sob):
        wid = lax.axis_index("s") * 2 + lax.axis_index("c")
        t0 = wid * TPW
        pltpu.sync_copy(pos0_hbm.at[pl.ds(t0, TPW)], idx0_v)
        pltpu.sync_copy(pos1_hbm.at[pl.ds(t0, TPW)], idx1_v)
        pltpu.sync_copy(w0b_hbm.at[pl.ds(t0, TPW)], w0_v)
        pltpu.sync_copy(w1b_hbm.at[pl.ds(t0, TPW)], w1_v)

        bufs = [(r0a, r1a, oa, s0a, s1a, soa), (r0b, r1b, ob, s0b, s1b, sob)]
        nch = TPW // CH

        def issue(c):
            r0, r1, _, sg0, sg1, _ = bufs[c & 1]
            i0 = idx0_v.at[pl.ds(c * CH, CH)]
            i1 = idx1_v.at[pl.ds(c * CH, CH)]
            return (pltpu.async_copy(ys_hbm.at[i0], r0, sg0),
                    pltpu.async_copy(ys_hbm.at[i1], r1, sg1))

        pend = {0: issue(0)}
        st = {}
        for c in range(nch):
            r0, r1, o_v, _, _, so = bufs[c & 1]
            if c + 1 < nch:
                pend[c + 1] = issue(c + 1)
            if c - 2 in st:
                st[c - 2].wait()
            pend[c][0].wait()
            pend[c][1].wait()
            for t in range(CH):
                w0v = w0_v[c * CH + t, :]
                w1v = w1_v[c * CH + t, :]

                def body(jj, carry, _t=t, _w0=w0v, _w1=w1v,
                         _r0=r0, _r1=r1, _o=o_v):
                    j0 = jj * 64
                    for u in range(4):
                        sl = pl.ds(j0 + u * 16, 16)
                        _o[_t, sl] = _w0 * _r0[_t, sl] + _w1 * _r1[_t, sl]
                    return carry

                lax.fori_loop(0, D // 64, body, 0)
            cp = pltpu.make_async_copy(
                o_v, out_hbm.at[pl.ds(t0 + c * CH, CH)], so)
            cp.start()
            st[c] = cp
        for c in (nch - 2, nch - 1):
            if c in st:
                st[c].wait()

    return _gather_sc, _combine_sc


def _ffn_body(be_ref, slot_ref, nxt_ref, xs_ref, w1_any, b1_ref, w2_any,
              b2_ref, out_ref, w1buf, w2buf, s1a, s1b, s2a, s2b):
    mi = pl.program_id(0)
    e = be_ref[mi]
    slot = slot_ref[mi]
    prev_e = be_ref[jnp.maximum(mi - 1, 0)]
    run_start = jnp.logical_or(mi == 0, e != prev_e)

    def w1_copy(src_e, dst_slot, sem):
        return pltpu.make_async_copy(
            w1_any.at[src_e], w1buf.at[pl.ds(dst_slot * D, D), :], sem)

    def w2_copy(src_e, dst_slot, sem):
        return pltpu.make_async_copy(
            w2_any.at[src_e], w2buf.at[pl.ds(dst_slot * H, H), :], sem)

    @pl.when(mi == 0)
    def _():
        w1_copy(e, 0, s1a).start()
        w2_copy(e, 0, s2a).start()

    ne = nxt_ref[mi]

    @pl.when(run_start & (ne >= 0))
    def _():
        @pl.when(slot == 1)
        def _():
            w1_copy(ne, 0, s1a).start()
            w2_copy(ne, 0, s2a).start()

        @pl.when(slot == 0)
        def _():
            w1_copy(ne, 1, s1b).start()
            w2_copy(ne, 1, s2b).start()

    @pl.when(e >= 0)
    def _():
        ec = jnp.maximum(e, 0)

        @pl.when(run_start & (slot == 0))
        def _():
            w1_copy(ec, 0, s1a).wait()

        @pl.when(run_start & (slot == 1))
        def _():
            w1_copy(ec, 1, s1b).wait()

        b1v = jnp.reshape(b1_ref[pl.ds(ec, 1)], (1, H))
        hmat = jnp.dot(xs_ref[...], w1buf[pl.ds(slot * D, D), :],
                       preferred_element_type=jnp.float32) + b1v
        g = 0.5 * hmat * (1.0 + lax.erf(hmat * (1.0 / math.sqrt(2.0))))

        @pl.when(run_start & (slot == 0))
        def _():
            w2_copy(ec, 0, s2a).wait()

        @pl.when(run_start & (slot == 1))
        def _():
            w2_copy(ec, 1, s2b).wait()

        b2v = jnp.reshape(b2_ref[pl.ds(ec, 1)], (1, D))
        out_ref[...] = jnp.dot(g, w2buf[pl.ds(slot * H, H), :],
                               preferred_element_type=jnp.float32) + b2v


_ffn = pl.pallas_call(
    _ffn_body,
    grid_spec=pltpu.PrefetchScalarGridSpec(
        num_scalar_prefetch=3,
        grid=(B_MAX,),
        in_specs=[
            pl.BlockSpec((BM, D), lambda mi, be, slot, nxt: (mi, 0)),
            pl.BlockSpec(memory_space=pl.ANY),
            pl.BlockSpec((E, 1, H), lambda mi, be, slot, nxt: (0, 0, 0)),
            pl.BlockSpec(memory_space=pl.ANY),
            pl.BlockSpec((E, 1, D), lambda mi, be, slot, nxt: (0, 0, 0)),
        ],
        out_specs=pl.BlockSpec((BM, D), lambda mi, be, slot, nxt: (mi, 0)),
        scratch_shapes=[pltpu.VMEM((2 * D, H), jnp.float32),
                        pltpu.VMEM((2 * H, D), jnp.float32),
                        pltpu.SemaphoreType.DMA,
                        pltpu.SemaphoreType.DMA,
                        pltpu.SemaphoreType.DMA,
                        pltpu.SemaphoreType.DMA],
    ),
    out_shape=jax.ShapeDtypeStruct((M_PAD, D), jnp.float32),
    compiler_params=pltpu.CompilerParams(
        dimension_semantics=("arbitrary",)),
)


@jax.jit
def kernel(x, router_W, W1, b1, W2, b2):
    gather_sc, combine_sc = _sc_kernels()
    xf = x.reshape(T, D)
    probs, pos0, pos1, w0b, w1b, be, slot, nxt = _router(xf, router_W)
    pos0 = pos0.reshape(T)
    pos1 = pos1.reshape(T)
    xs = gather_sc(xf, pos0, pos1)
    ys = _ffn(be.reshape(B_MAX), slot.reshape(B_MAX), nxt.reshape(B_MAX),
              xs, W1, b1.reshape(E, 1, H), W2, b2.reshape(E, 1, D))
    out = combine_sc(ys, pos0, pos1, w0b, w1b)
    return out.reshape(x.shape), probs


# speedup vs baseline: 1.1613x; 1.1613x over previous
import functools
import math

import jax
import jax.numpy as jnp
from jax import lax
from jax.experimental import pallas as pl
from jax.experimental.pallas import tpu as pltpu
from jax.experimental.pallas import tpu_sc as plsc

T = 2048
D = 1024
H = 2048
E = 8
BM = 256
BM_LOG2 = 8
B_MAX = (T * 2) // BM + E
M_PAD = B_MAX * BM
BH = 512
NH = H // BH
NW = 32
TPW = T // NW
CH = 16


def _router_body(x_ref, rw_ref, probs_ref, pos0_ref, pos1_ref,
                 w0b_ref, w1b_ref, be_ref, slot_ref, nxt_ref):
    x = x_ref[...]
    logits = lax.dot_general(x, rw_ref[...], (((1,), (1,)), ((), ())),
                             preferred_element_type=jnp.float32)
    mx = jnp.max(logits, axis=1, keepdims=True)
    ex = jnp.exp(logits - mx)
    probs = ex / jnp.sum(ex, axis=1, keepdims=True)
    probs_ref[...] = probs

    e_io = lax.broadcasted_iota(jnp.int32, (T, E), 1)
    v1 = jnp.max(probs, axis=1, keepdims=True)
    i1 = jnp.min(jnp.where(probs == v1, e_io, E), axis=1, keepdims=True)
    pmask = jnp.where(e_io == i1, -1.0, probs)
    v2 = jnp.max(pmask, axis=1, keepdims=True)
    i2 = jnp.min(jnp.where(pmask == v2, e_io, E), axis=1, keepdims=True)
    den = v1 + v2
    w0b_ref[...] = jnp.broadcast_to(v1 / den, (T, 16))
    w1b_ref[...] = jnp.broadcast_to(v2 / den, (T, 16))

    h1 = (e_io == i1).astype(jnp.float32)
    h2 = (e_io == i2).astype(jnp.float32)
    eye = (lax.broadcasted_iota(jnp.int32, (E, E), 0) ==
           lax.broadcasted_iota(jnp.int32, (E, E), 1)).astype(jnp.float32)
    tdims = (((1,), (1,)), ((), ()))
    g1 = lax.dot_general(eye, h1, tdims, preferred_element_type=jnp.float32)
    g2 = lax.dot_general(eye, h2, tdims, preferred_element_type=jnp.float32)

    def cumsum_lanes(g):
        s = 1
        while s < T:
            g = g + jnp.concatenate(
                [jnp.zeros((E, s), jnp.float32), g[:, :T - s]], axis=1)
            s *= 2
        return g

    c1 = cumsum_lanes(g1)
    c2 = cumsum_lanes(g2)
    c1t = c1[:, T - 1:T]
    c2t = c2[:, T - 1:T]
    counts = (c1t + c2t).astype(jnp.int32)
    nb = lax.shift_right_logical(counts + (BM - 1), BM_LOG2)

    def cumsum_sub(v):
        s = 1
        while s < E:
            v = v + jnp.concatenate(
                [jnp.zeros((s, 1), v.dtype), v[:E - s, :]], axis=0)
            s *= 2
        return v

    off_blk = cumsum_sub(nb) - nb
    off_pad = (off_blk * BM).astype(jnp.float32)
    r0 = c1 - g1
    r1 = c1t + c2 - g2
    pos0_ref[...] = jnp.sum(g1 * (off_pad + r0), axis=0,
                            keepdims=True).astype(jnp.int32)
    pos1_ref[...] = jnp.sum(g2 * (off_pad + r1), axis=0,
                            keepdims=True).astype(jnp.int32)

    m_io = lax.broadcasted_iota(jnp.int32, (E, B_MAX), 1)
    lo = jnp.broadcast_to(off_blk, (E, B_MAX))
    hi = lo + jnp.broadcast_to(nb, (E, B_MAX))
    e_col = lax.broadcasted_iota(jnp.int32, (E, B_MAX), 0)
    ind01 = ((m_io >= lo) & (m_io < hi)).astype(jnp.int32)
    bev = jnp.sum(ind01 * (e_col + 1), axis=0, keepdims=True) - 1
    be_ref[...] = bev

    be_prev = jnp.concatenate(
        [jnp.full((1, 1), -7, jnp.int32), bev[:, :B_MAX - 1]], axis=1)
    active = bev >= 0
    run_start = (active & (bev != be_prev)).astype(jnp.float32)
    s = 1
    while s < B_MAX:
        run_start = run_start + jnp.concatenate(
            [jnp.zeros((1, s), jnp.float32), run_start[:, :B_MAX - s]], axis=1)
        s *= 2
    slot_ref[...] = jnp.bitwise_and(run_start.astype(jnp.int32) - 1, 1)

    nbm = jnp.sum(ind01 * jnp.broadcast_to(nb, (E, B_MAX)), axis=0,
                  keepdims=True)
    m_row = lax.broadcasted_iota(jnp.int32, (1, B_MAX), 1)
    nxt_idx = m_row + nbm
    tb = off_blk[E - 1:E, :] + nb[E - 1:E, :]
    ci = jnp.minimum(nxt_idx, B_MAX - 1)
    pmat = (lax.broadcasted_iota(jnp.int32, (B_MAX, B_MAX), 0) ==
            jnp.broadcast_to(ci, (B_MAX, B_MAX))).astype(jnp.float32)
    nxtv = lax.dot_general(bev.astype(jnp.float32), pmat,
                           (((1,), (0,)), ((), ())),
                           preferred_element_type=jnp.float32)
    nxt_ref[...] = jnp.where(active & (nxt_idx < tb),
                             nxtv.astype(jnp.int32), -1)


_router = pl.pallas_call(
    _router_body,
    out_shape=[
        jax.ShapeDtypeStruct((T, E), jnp.float32),
        jax.ShapeDtypeStruct((1, T), jnp.int32),
        jax.ShapeDtypeStruct((1, T), jnp.int32),
        jax.ShapeDtypeStruct((T, 16), jnp.float32),
        jax.ShapeDtypeStruct((T, 16), jnp.float32),
        jax.ShapeDtypeStruct((1, B_MAX), jnp.int32),
        jax.ShapeDtypeStruct((1, B_MAX), jnp.int32),
        jax.ShapeDtypeStruct((1, B_MAX), jnp.int32),
    ],
)


@functools.cache
def _sc_kernels():
    mesh = plsc.VectorSubcoreMesh(core_axis_name="c", subcore_axis_name="s")

    @functools.partial(
        pl.kernel,
        mesh=mesh,
        out_type=jax.ShapeDtypeStruct((M_PAD, D), jnp.float32),
        scratch_types=[
            pltpu.VMEM((TPW,), jnp.int32),
            pltpu.VMEM((TPW,), jnp.int32),
            pltpu.VMEM((TPW, D), jnp.float32),
            pltpu.SemaphoreType.DMA,
        ],
    )
    def _gather_sc(x_hbm, pos0_hbm, pos1_hbm, xs_hbm,
                   idx0_v, idx1_v, rows_v, sem):
        wid = lax.axis_index("s") * 2 + lax.axis_index("c")
        t0 = wid * TPW
        pltpu.sync_copy(pos0_hbm.at[pl.ds(t0, TPW)], idx0_v)
        pltpu.sync_copy(pos1_hbm.at[pl.ds(t0, TPW)], idx1_v)
        pltpu.sync_copy(x_hbm.at[pl.ds(t0, TPW)], rows_v)
        pltpu.async_copy(rows_v, xs_hbm.at[idx0_v], sem).wait()
        pltpu.async_copy(rows_v, xs_hbm.at[idx1_v], sem).wait()

    @functools.partial(
        pl.kernel,
        mesh=mesh,
        out_type=jax.ShapeDtypeStruct((T, D), jnp.float32),
        scratch_types=[
            pltpu.VMEM((TPW,), jnp.int32),
            pltpu.VMEM((TPW,), jnp.int32),
            pltpu.VMEM((TPW, 16), jnp.float32),
            pltpu.VMEM((TPW, 16), jnp.float32),
            pltpu.VMEM((CH, D), jnp.float32),
            pltpu.VMEM((CH, D), jnp.float32),
            pltpu.VMEM((CH, D), jnp.float32),
            pltpu.VMEM((CH, D), jnp.float32),
            pltpu.VMEM((CH, D), jnp.float32),
            pltpu.VMEM((CH, D), jnp.float32),
            pltpu.SemaphoreType.DMA,
            pltpu.SemaphoreType.DMA,
            pltpu.SemaphoreType.DMA,
            pltpu.SemaphoreType.DMA,
            pltpu.SemaphoreType.DMA,
            pltpu.SemaphoreType.DMA,
        ],
    )
    def _combine_sc(ys_hbm, pos0_hbm, pos1_hbm, w0b_hbm, w1b_hbm, out_hbm,
                    idx0_v, idx1_v, w0_v, w1_v,
                    r0a, r1a, oa, r0b, r1b, ob,
                    s0a, s1a, soa, s0b, s1b, sob):
        wid = lax.axis_index("s") * 2 + lax.axis_index("c")
        t0 = wid * TPW
        pltpu.sync_copy(pos0_hbm.at[pl.ds(t0, TPW)], idx0_v)
        pltpu.sync_copy(pos1_hbm.at[pl.ds(t0, TPW)], idx1_v)
        pltpu.sync_copy(w0b_hbm.at[pl.ds(t0, TPW)], w0_v)
        pltpu.sync_copy(w1b_hbm.at[pl.ds(t0, TPW)], w1_v)

        bufs = [(r0a, r1a, oa, s0a, s1a, soa), (r0b, r1b, ob, s0b, s1b, sob)]
        nch = TPW // CH

        def issue(c):
            r0, r1, _, sg0, sg1, _ = bufs[c & 1]
            i0 = idx0_v.at[pl.ds(c * CH, CH)]
            i1 = idx1_v.at[pl.ds(c * CH, CH)]
            return (pltpu.async_copy(ys_hbm.at[i0], r0, sg0),
                    pltpu.async_copy(ys_hbm.at[i1], r1, sg1))

        pend = {0: issue(0)}
        st = {}
        for c in range(nch):
            r0, r1, o_v, _, _, so = bufs[c & 1]
            if c + 1 < nch:
                pend[c + 1] = issue(c + 1)
            if c - 2 in st:
                st[c - 2].wait()
            pend[c][0].wait()
            pend[c][1].wait()
            for t in range(CH):
                w0v = w0_v[c * CH + t, :]
                w1v = w1_v[c * CH + t, :]

                def body(jj, carry, _t=t, _w0=w0v, _w1=w1v,
                         _r0=r0, _r1=r1, _o=o_v):
                    j0 = jj * 64
                    for u in range(4):
                        sl = pl.ds(j0 + u * 16, 16)
                        _o[_t, sl] = _w0 * _r0[_t, sl] + _w1 * _r1[_t, sl]
                    return carry

                lax.fori_loop(0, D // 64, body, 0)
            cp = pltpu.make_async_copy(
                o_v, out_hbm.at[pl.ds(t0 + c * CH, CH)], so)
            cp.start()
            st[c] = cp
        for c in (nch - 2, nch - 1):
            if c in st:
                st[c].wait()

    return _gather_sc, _combine_sc


def _ffn_body(be_ref, slot_ref, nxt_ref, xs_ref, w1_any, b1_ref, w2_any,
              b2_ref, out_ref, w1buf, w2buf, s1a, s1b, s2a, s2b):
    mi = pl.program_id(0)
    e = be_ref[mi]
    slot = slot_ref[mi]
    prev_e = be_ref[jnp.maximum(mi - 1, 0)]
    run_start = jnp.logical_or(mi == 0, e != prev_e)

    def w1_copy(src_e, dst_slot, sem):
        return pltpu.make_async_copy(
            w1_any.at[src_e], w1buf.at[pl.ds(dst_slot * D, D), :], sem)

    def w2_copy(src_e, dst_slot, sem):
        return pltpu.make_async_copy(
            w2_any.at[src_e], w2buf.at[pl.ds(dst_slot * H, H), :], sem)

    @pl.when(mi == 0)
    def _():
        w1_copy(e, 0, s1a).start()
        w2_copy(e, 0, s2a).start()

    ne = nxt_ref[mi]

    @pl.when(run_start & (ne >= 0))
    def _():
        @pl.when(slot == 1)
        def _():
            w1_copy(ne, 0, s1a).start()
            w2_copy(ne, 0, s2a).start()

        @pl.when(slot == 0)
        def _():
            w1_copy(ne, 1, s1b).start()
            w2_copy(ne, 1, s2b).start()

    @pl.when(e >= 0)
    def _():
        ec = jnp.maximum(e, 0)

        @pl.when(run_start & (slot == 0))
        def _():
            w1_copy(ec, 0, s1a).wait()

        @pl.when(run_start & (slot == 1))
        def _():
            w1_copy(ec, 1, s1b).wait()

        b1v = jnp.reshape(b1_ref[pl.ds(ec, 1)], (1, H))
        hmat = jnp.dot(xs_ref[...], w1buf[pl.ds(slot * D, D), :],
                       preferred_element_type=jnp.float32) + b1v
        g = 0.5 * hmat * (1.0 + lax.erf(hmat * (1.0 / math.sqrt(2.0))))

        @pl.when(run_start & (slot == 0))
        def _():
            w2_copy(ec, 0, s2a).wait()

        @pl.when(run_start & (slot == 1))
        def _():
            w2_copy(ec, 1, s2b).wait()

        b2v = jnp.reshape(b2_ref[pl.ds(ec, 1)], (1, D))
        out_ref[...] = jnp.dot(g, w2buf[pl.ds(slot * H, H), :],
                               preferred_element_type=jnp.float32) + b2v


_ffn = pl.pallas_call(
    _ffn_body,
    grid_spec=pltpu.PrefetchScalarGridSpec(
        num_scalar_prefetch=3,
        grid=(B_MAX,),
        in_specs=[
            pl.BlockSpec((BM, D), lambda mi, be, slot, nxt: (mi, 0)),
            pl.BlockSpec(memory_space=pl.ANY),
            pl.BlockSpec((E, 1, H), lambda mi, be, slot, nxt: (0, 0, 0)),
            pl.BlockSpec(memory_space=pl.ANY),
            pl.BlockSpec((E, 1, D), lambda mi, be, slot, nxt: (0, 0, 0)),
        ],
        out_specs=pl.BlockSpec((BM, D), lambda mi, be, slot, nxt: (mi, 0)),
        scratch_shapes=[pltpu.VMEM((2 * D, H), jnp.float32),
                        pltpu.VMEM((2 * H, D), jnp.float32),
                        pltpu.SemaphoreType.DMA,
                        pltpu.SemaphoreType.DMA,
                        pltpu.SemaphoreType.DMA,
                        pltpu.SemaphoreType.DMA],
    ),
    out_shape=jax.ShapeDtypeStruct((M_PAD, D), jnp.float32),
    compiler_params=pltpu.CompilerParams(
        dimension_semantics=("arbitrary",)),
)


@jax.jit
def kernel(x, router_W, W1, b1, W2, b2):
    gather_sc, combine_sc = _sc_kernels()
    xf = x.reshape(T, D)
    probs, pos0, pos1, w0b, w1b, be, slot, nxt = _router(xf, router_W)
    pos0 = pos0.reshape(T)
    pos1 = pos1.reshape(T)
    xs = gather_sc(xf, pos0, pos1)
    ys = _ffn(be.reshape(B_MAX), slot.reshape(B_MAX), nxt.reshape(B_MAX),
              xs, W1, b1.reshape(E, 1, H), W2, b2.reshape(E, 1, D))
    out = combine_sc(ys, pos0, pos1, w0b, w1b)
    return out.reshape(x.shape), probs


# speedup vs baseline: 1.1832x; 1.0189x over previous
import functools
import math

import jax
import jax.numpy as jnp
from jax import lax
from jax.experimental import pallas as pl
from jax.experimental.pallas import tpu as pltpu
from jax.experimental.pallas import tpu_sc as plsc

T = 2048
D = 1024
H = 2048
E = 8
BM = 256
BM_LOG2 = 8
B_MAX = (T * 2) // BM + E
M_PAD = B_MAX * BM
BH = 512
NH = H // BH
NW = 32
TPW = T // NW
CH = 16


def _router_body(x_ref, rw_ref, probs_ref, pos0_ref, pos1_ref,
                 w0b_ref, w1b_ref, be_ref, slot_ref, nxt_ref):
    x = x_ref[...]
    logits = lax.dot_general(x, rw_ref[...], (((1,), (1,)), ((), ())),
                             preferred_element_type=jnp.float32)
    mx = jnp.max(logits, axis=1, keepdims=True)
    ex = jnp.exp(logits - mx)
    probs = ex / jnp.sum(ex, axis=1, keepdims=True)
    probs_ref[...] = probs

    e_io = lax.broadcasted_iota(jnp.int32, (T, E), 1)
    v1 = jnp.max(probs, axis=1, keepdims=True)
    i1 = jnp.min(jnp.where(probs == v1, e_io, E), axis=1, keepdims=True)
    pmask = jnp.where(e_io == i1, -1.0, probs)
    v2 = jnp.max(pmask, axis=1, keepdims=True)
    i2 = jnp.min(jnp.where(pmask == v2, e_io, E), axis=1, keepdims=True)
    den = v1 + v2
    w0b_ref[...] = jnp.broadcast_to(v1 / den, (T, 16))
    w1b_ref[...] = jnp.broadcast_to(v2 / den, (T, 16))

    h1 = (e_io == i1).astype(jnp.float32)
    h2 = (e_io == i2).astype(jnp.float32)
    eye = (lax.broadcasted_iota(jnp.int32, (E, E), 0) ==
           lax.broadcasted_iota(jnp.int32, (E, E), 1)).astype(jnp.float32)
    tdims = (((1,), (1,)), ((), ()))
    g1 = lax.dot_general(eye, h1, tdims, preferred_element_type=jnp.float32)
    g2 = lax.dot_general(eye, h2, tdims, preferred_element_type=jnp.float32)

    def cumsum_lanes(g):
        s = 1
        while s < T:
            g = g + jnp.concatenate(
                [jnp.zeros((E, s), jnp.float32), g[:, :T - s]], axis=1)
            s *= 2
        return g

    c1 = cumsum_lanes(g1)
    c2 = cumsum_lanes(g2)
    c1t = c1[:, T - 1:T]
    c2t = c2[:, T - 1:T]
    counts = (c1t + c2t).astype(jnp.int32)
    nb = lax.shift_right_logical(counts + (BM - 1), BM_LOG2)

    def cumsum_sub(v):
        s = 1
        while s < E:
            v = v + jnp.concatenate(
                [jnp.zeros((s, 1), v.dtype), v[:E - s, :]], axis=0)
            s *= 2
        return v

    off_blk = cumsum_sub(nb) - nb
    off_pad = (off_blk * BM).astype(jnp.float32)
    r0 = c1 - g1
    r1 = c1t + c2 - g2
    pos0_ref[...] = jnp.sum(g1 * (off_pad + r0), axis=0,
                            keepdims=True).astype(jnp.int32)
    pos1_ref[...] = jnp.sum(g2 * (off_pad + r1), axis=0,
                            keepdims=True).astype(jnp.int32)

    m_io = lax.broadcasted_iota(jnp.int32, (E, B_MAX), 1)
    lo = jnp.broadcast_to(off_blk, (E, B_MAX))
    hi = lo + jnp.broadcast_to(nb, (E, B_MAX))
    e_col = lax.broadcasted_iota(jnp.int32, (E, B_MAX), 0)
    ind01 = ((m_io >= lo) & (m_io < hi)).astype(jnp.int32)
    bev = jnp.sum(ind01 * (e_col + 1), axis=0, keepdims=True) - 1
    be_ref[...] = bev

    be_prev = jnp.concatenate(
        [jnp.full((1, 1), -7, jnp.int32), bev[:, :B_MAX - 1]], axis=1)
    active = bev >= 0
    run_start = (active & (bev != be_prev)).astype(jnp.float32)
    s = 1
    while s < B_MAX:
        run_start = run_start + jnp.concatenate(
            [jnp.zeros((1, s), jnp.float32), run_start[:, :B_MAX - s]], axis=1)
        s *= 2
    slot_ref[...] = jnp.bitwise_and(run_start.astype(jnp.int32) - 1, 1)

    nbm = jnp.sum(ind01 * jnp.broadcast_to(nb, (E, B_MAX)), axis=0,
                  keepdims=True)
    m_row = lax.broadcasted_iota(jnp.int32, (1, B_MAX), 1)
    nxt_idx = m_row + nbm
    tb = off_blk[E - 1:E, :] + nb[E - 1:E, :]
    ci = jnp.minimum(nxt_idx, B_MAX - 1)
    pmat = (lax.broadcasted_iota(jnp.int32, (B_MAX, B_MAX), 0) ==
            jnp.broadcast_to(ci, (B_MAX, B_MAX))).astype(jnp.float32)
    nxtv = lax.dot_general(bev.astype(jnp.float32), pmat,
                           (((1,), (0,)), ((), ())),
                           preferred_element_type=jnp.float32)
    nxt_ref[...] = jnp.where(active & (nxt_idx < tb),
                             nxtv.astype(jnp.int32), -1)


_router = pl.pallas_call(
    _router_body,
    out_shape=[
        jax.ShapeDtypeStruct((T, E), jnp.float32),
        jax.ShapeDtypeStruct((1, T), jnp.int32),
        jax.ShapeDtypeStruct((1, T), jnp.int32),
        jax.ShapeDtypeStruct((T, 16), jnp.float32),
        jax.ShapeDtypeStruct((T, 16), jnp.float32),
        jax.ShapeDtypeStruct((1, B_MAX), jnp.int32),
        jax.ShapeDtypeStruct((1, B_MAX), jnp.int32),
        jax.ShapeDtypeStruct((1, B_MAX), jnp.int32),
    ],
)


@functools.cache
def _sc_kernels():
    mesh = plsc.VectorSubcoreMesh(core_axis_name="c", subcore_axis_name="s")

    @functools.partial(
        pl.kernel,
        mesh=mesh,
        out_type=jax.ShapeDtypeStruct((M_PAD, D), jnp.float32),
        scratch_types=[
            pltpu.VMEM((TPW,), jnp.int32),
            pltpu.VMEM((TPW,), jnp.int32),
            pltpu.VMEM((TPW, D), jnp.float32),
            pltpu.SemaphoreType.DMA,
        ],
    )
    def _gather_sc(x_hbm, pos0_hbm, pos1_hbm, xs_hbm,
                   idx0_v, idx1_v, rows_v, sem):
        wid = lax.axis_index("s") * 2 + lax.axis_index("c")
        t0 = wid * TPW
        pltpu.sync_copy(pos0_hbm.at[pl.ds(t0, TPW)], idx0_v)
        pltpu.sync_copy(pos1_hbm.at[pl.ds(t0, TPW)], idx1_v)
        pltpu.sync_copy(x_hbm.at[pl.ds(t0, TPW)], rows_v)
        cp0 = pltpu.async_copy(rows_v, xs_hbm.at[idx0_v], sem)
        cp1 = pltpu.async_copy(rows_v, xs_hbm.at[idx1_v], sem)
        cp0.wait()
        cp1.wait()

    @functools.partial(
        pl.kernel,
        mesh=mesh,
        out_type=jax.ShapeDtypeStruct((T, D), jnp.float32),
        scratch_types=[
            pltpu.VMEM((TPW,), jnp.int32),
            pltpu.VMEM((TPW,), jnp.int32),
            pltpu.VMEM((TPW, 16), jnp.float32),
            pltpu.VMEM((TPW, 16), jnp.float32),
            pltpu.VMEM((CH, D), jnp.float32),
            pltpu.VMEM((CH, D), jnp.float32),
            pltpu.VMEM((CH, D), jnp.float32),
            pltpu.VMEM((CH, D), jnp.float32),
            pltpu.VMEM((CH, D), jnp.float32),
            pltpu.VMEM((CH, D), jnp.float32),
            pltpu.SemaphoreType.DMA,
            pltpu.SemaphoreType.DMA,
            pltpu.SemaphoreType.DMA,
            pltpu.SemaphoreType.DMA,
            pltpu.SemaphoreType.DMA,
            pltpu.SemaphoreType.DMA,
        ],
    )
    def _combine_sc(ys_hbm, pos0_hbm, pos1_hbm, w0b_hbm, w1b_hbm, out_hbm,
                    idx0_v, idx1_v, w0_v, w1_v,
                    r0a, r1a, oa, r0b, r1b, ob,
                    s0a, s1a, soa, s0b, s1b, sob):
        wid = lax.axis_index("s") * 2 + lax.axis_index("c")
        t0 = wid * TPW
        pltpu.sync_copy(pos0_hbm.at[pl.ds(t0, TPW)], idx0_v)
        pltpu.sync_copy(pos1_hbm.at[pl.ds(t0, TPW)], idx1_v)
        pltpu.sync_copy(w0b_hbm.at[pl.ds(t0, TPW)], w0_v)
        pltpu.sync_copy(w1b_hbm.at[pl.ds(t0, TPW)], w1_v)

        bufs = [(r0a, r1a, oa, s0a, s1a, soa), (r0b, r1b, ob, s0b, s1b, sob)]
        nch = TPW // CH

        def issue(c):
            r0, r1, _, sg0, sg1, _ = bufs[c & 1]
            i0 = idx0_v.at[pl.ds(c * CH, CH)]
            i1 = idx1_v.at[pl.ds(c * CH, CH)]
            return (pltpu.async_copy(ys_hbm.at[i0], r0, sg0),
                    pltpu.async_copy(ys_hbm.at[i1], r1, sg1))

        pend = {0: issue(0)}
        st = {}
        for c in range(nch):
            r0, r1, o_v, _, _, so = bufs[c & 1]
            if c + 1 < nch:
                pend[c + 1] = issue(c + 1)
            if c - 2 in st:
                st[c - 2].wait()
            pend[c][0].wait()
            pend[c][1].wait()
            for t in range(CH):
                w0v = w0_v[c * CH + t, :]
                w1v = w1_v[c * CH + t, :]

                def body(jj, carry, _t=t, _w0=w0v, _w1=w1v,
                         _r0=r0, _r1=r1, _o=o_v):
                    j0 = jj * 256
                    for u in range(16):
                        sl = pl.ds(j0 + u * 16, 16)
                        _o[_t, sl] = _w0 * _r0[_t, sl] + _w1 * _r1[_t, sl]
                    return carry

                lax.fori_loop(0, D // 256, body, 0)
            cp = pltpu.make_async_copy(
                o_v, out_hbm.at[pl.ds(t0 + c * CH, CH)], so)
            cp.start()
            st[c] = cp
        for c in (nch - 2, nch - 1):
            if c in st:
                st[c].wait()

    return _gather_sc, _combine_sc


def _ffn_body(be_ref, slot_ref, nxt_ref, xs_ref, w1_any, b1_ref, w2_any,
              b2_ref, out_ref, w1buf, w2buf, s1a, s1b, s2a, s2b):
    mi = pl.program_id(0)
    e = be_ref[mi]
    slot = slot_ref[mi]
    prev_e = be_ref[jnp.maximum(mi - 1, 0)]
    run_start = jnp.logical_or(mi == 0, e != prev_e)

    def w1_copy(src_e, dst_slot, sem):
        return pltpu.make_async_copy(
            w1_any.at[src_e], w1buf.at[pl.ds(dst_slot * D, D), :], sem)

    def w2_copy(src_e, dst_slot, sem):
        return pltpu.make_async_copy(
            w2_any.at[src_e], w2buf.at[pl.ds(dst_slot * H, H), :], sem)

    @pl.when(mi == 0)
    def _():
        w1_copy(e, 0, s1a).start()
        w2_copy(e, 0, s2a).start()

    ne = nxt_ref[mi]

    @pl.when(run_start & (ne >= 0))
    def _():
        @pl.when(slot == 1)
        def _():
            w1_copy(ne, 0, s1a).start()
            w2_copy(ne, 0, s2a).start()

        @pl.when(slot == 0)
        def _():
            w1_copy(ne, 1, s1b).start()
            w2_copy(ne, 1, s2b).start()

    @pl.when(e >= 0)
    def _():
        ec = jnp.maximum(e, 0)

        @pl.when(run_start & (slot == 0))
        def _():
            w1_copy(ec, 0, s1a).wait()

        @pl.when(run_start & (slot == 1))
        def _():
            w1_copy(ec, 1, s1b).wait()

        b1v = jnp.reshape(b1_ref[pl.ds(ec, 1)], (1, H))
        hmat = jnp.dot(xs_ref[...], w1buf[pl.ds(slot * D, D), :],
                       preferred_element_type=jnp.float32) + b1v
        g = 0.5 * hmat * (1.0 + lax.erf(hmat * (1.0 / math.sqrt(2.0))))

        @pl.when(run_start & (slot == 0))
        def _():
            w2_copy(ec, 0, s2a).wait()

        @pl.when(run_start & (slot == 1))
        def _():
            w2_copy(ec, 1, s2b).wait()

        b2v = jnp.reshape(b2_ref[pl.ds(ec, 1)], (1, D))
        out_ref[...] = jnp.dot(g, w2buf[pl.ds(slot * H, H), :],
                               preferred_element_type=jnp.float32) + b2v


_ffn = pl.pallas_call(
    _ffn_body,
    grid_spec=pltpu.PrefetchScalarGridSpec(
        num_scalar_prefetch=3,
        grid=(B_MAX,),
        in_specs=[
            pl.BlockSpec((BM, D), lambda mi, be, slot, nxt: (mi, 0)),
            pl.BlockSpec(memory_space=pl.ANY),
            pl.BlockSpec((E, 1, H), lambda mi, be, slot, nxt: (0, 0, 0)),
            pl.BlockSpec(memory_space=pl.ANY),
            pl.BlockSpec((E, 1, D), lambda mi, be, slot, nxt: (0, 0, 0)),
        ],
        out_specs=pl.BlockSpec((BM, D), lambda mi, be, slot, nxt: (mi, 0)),
        scratch_shapes=[pltpu.VMEM((2 * D, H), jnp.float32),
                        pltpu.VMEM((2 * H, D), jnp.float32),
                        pltpu.SemaphoreType.DMA,
                        pltpu.SemaphoreType.DMA,
                        pltpu.SemaphoreType.DMA,
                        pltpu.SemaphoreType.DMA],
    ),
    out_shape=jax.ShapeDtypeStruct((M_PAD, D), jnp.float32),
    compiler_params=pltpu.CompilerParams(
        dimension_semantics=("arbitrary",)),
)


@jax.jit
def kernel(x, router_W, W1, b1, W2, b2):
    gather_sc, combine_sc = _sc_kernels()
    xf = x.reshape(T, D)
    probs, pos0, pos1, w0b, w1b, be, slot, nxt = _router(xf, router_W)
    pos0 = pos0.reshape(T)
    pos1 = pos1.reshape(T)
    xs = gather_sc(xf, pos0, pos1)
    ys = _ffn(be.reshape(B_MAX), slot.reshape(B_MAX), nxt.reshape(B_MAX),
              xs, W1, b1.reshape(E, 1, H), W2, b2.reshape(E, 1, D))
    out = combine_sc(ys, pos0, pos1, w0b, w1b)
    return out.reshape(x.shape), probs
